# baseline (speedup 1.0000x reference)
import functools

import jax
import jax.numpy as jnp
from jax import lax
from jax.experimental import pallas as pl
from jax.experimental.pallas import tpu as pltpu
from jax.experimental.pallas import tpu_sc as plsc


def _head_body(nsub, nk, *refs):
    x_ref = refs[0]
    e_refs = refs[1 : 1 + nsub]
    emb_ref = refs[1 + nsub]
    idx_ref = refs[2 + nsub]
    best_ref = refs[3 + nsub]
    bidx_ref = refs[4 + nsub]

    k = pl.program_id(1)
    kt = emb_ref.shape[0]

    x = x_ref[...]
    for e_ref in e_refs:
        x = x - e_ref[...].astype(jnp.bfloat16).astype(jnp.float32)
    sx = jnp.sum(x * x, axis=1)

    w = emb_ref[...]
    wsq = jnp.sum(w * w, axis=1)
    mm = lax.dot_general(
        x.astype(jnp.bfloat16),
        w.astype(jnp.bfloat16),
        dimension_numbers=(((1,), (1,)), ((), ())),
        preferred_element_type=jnp.float32,
    )
    t = (sx[:, None] + wsq[None, :]) - 2.0 * mm

    loc_min = jnp.min(t, axis=1)
    loc_arg = jnp.argmin(t, axis=1).astype(jnp.int32) + k * kt

    @pl.when(k == 0)
    def _init():
        best_ref[0, :] = loc_min
        bidx_ref[0, :] = loc_arg

    @pl.when(k > 0)
    def _update():
        prev = best_ref[0, :]
        upd = loc_min < prev
        best_ref[0, :] = jnp.where(upd, loc_min, prev)
        bidx_ref[0, :] = jnp.where(upd, loc_arg, bidx_ref[0, :])

    @pl.when(k == nk - 1)
    def _emit():
        idx_ref[0, 0, :] = bidx_ref[0, :]


def _head_argmin(emb, x, *es, bt=512, kt=1024):
    b, d = x.shape
    kk = emb.shape[0]
    nb = b // bt
    nk = kk // kt
    nsub = len(es)

    in_specs = [pl.BlockSpec((bt, d), lambda i, k: (i, 0))]
    for _ in range(nsub):
        in_specs.append(pl.BlockSpec((bt, d), lambda i, k: (i, 0)))
    in_specs.append(pl.BlockSpec((kt, d), lambda i, k: (k, 0)))

    out = pl.pallas_call(
        functools.partial(_head_body, nsub, nk),
        grid=(nb, nk),
        in_specs=in_specs,
        out_specs=pl.BlockSpec((1, 1, bt), lambda i, k: (i, 0, 0)),
        out_shape=jax.ShapeDtypeStruct((nb, 1, bt), jnp.int32),
        scratch_shapes=[
            pltpu.VMEM((1, bt), jnp.float32),
            pltpu.VMEM((1, bt), jnp.int32),
        ],
        compiler_params=pltpu.CompilerParams(
            dimension_semantics=("parallel", "arbitrary"),
        ),
    )(x, *es, emb)
    return out.reshape(b)


def _sc_gather(table, idx):
    kk, d = table.shape
    b = idx.shape[0]
    info = plsc.get_sparse_core_info()
    nw = info.num_cores * info.num_subcores
    b_per_w = b // nw
    mesh = plsc.VectorSubcoreMesh(core_axis_name="c", subcore_axis_name="s")

    @functools.partial(
        pl.kernel,
        mesh=mesh,
        out_type=jax.ShapeDtypeStruct((b, d), jnp.float32),
        scratch_types=[
            pltpu.VMEM((b_per_w,), jnp.int32),
            pltpu.VMEM((b_per_w, d), jnp.float32),
            pltpu.SemaphoreType.DMA,
        ],
    )
    def gather(table_hbm, idx_hbm, out_hbm, idx_v, rows_v, sem):
        wid = lax.axis_index("s") * info.num_cores + lax.axis_index("c")
        base = wid * b_per_w
        pltpu.sync_copy(idx_hbm.at[pl.ds(base, b_per_w)], idx_v)
        pltpu.async_copy(table_hbm.at[idx_v], rows_v, sem).wait()
        pltpu.sync_copy(rows_v, out_hbm.at[pl.ds(base, b_per_w)])

    return gather(table, idx)


def _combine_body(e0_ref, e1_ref, e2_ref, out_ref):
    out_ref[...] = e0_ref[...] + e1_ref[...] + e2_ref[...]


def _combine(e0, e1, e2, bt=512):
    b, d = e0.shape
    nb = b // bt
    spec = pl.BlockSpec((bt, d), lambda i: (i, 0))
    return pl.pallas_call(
        _combine_body,
        grid=(nb,),
        in_specs=[spec, spec, spec],
        out_specs=spec,
        out_shape=jax.ShapeDtypeStruct((b, d), jnp.float32),
    )(e0, e1, e2)


def kernel(inputs, emb0, emb1, emb2):
    b = inputs.shape[0]
    x0 = inputs[:, 0, :]

    idx0 = _head_argmin(emb0, x0)
    e0 = _sc_gather(emb0, idx0)
    idx1 = _head_argmin(emb1, x0, e0)
    e1 = _sc_gather(emb1, idx1)
    idx2 = _head_argmin(emb2, x0, e0, e1)
    e2 = _sc_gather(emb2, idx2)

    quantized = _combine(e0, e1, e2).reshape(b, 1, inputs.shape[2])
    codes = jnp.stack([idx0, idx1, idx2], axis=1)
    return quantized, codes


# speedup vs baseline: 1.6598x; 1.6598x over previous
import functools

import jax
import jax.numpy as jnp
from jax import lax
from jax.experimental import pallas as pl
from jax.experimental.pallas import tpu as pltpu
from jax.experimental.pallas import tpu_sc as plsc


def _head_body(nsub, nk, *refs):
    x_ref = refs[0]
    e_refs = refs[1 : 1 + nsub]
    emb_ref = refs[1 + nsub]
    idx_ref = refs[2 + nsub]
    best_ref = refs[3 + nsub]
    bidx_ref = refs[4 + nsub]

    k = pl.program_id(1)
    kt = emb_ref.shape[0]

    x = x_ref[...]
    for e_ref in e_refs:
        x = x - e_ref[...].astype(jnp.bfloat16).astype(jnp.float32)
    sx = jnp.sum(x * x, axis=1)

    w = emb_ref[...]
    wsq = jnp.sum(w * w, axis=1)
    mm_t = lax.dot_general(
        w.astype(jnp.bfloat16),
        x.astype(jnp.bfloat16),
        dimension_numbers=(((1,), (1,)), ((), ())),
        preferred_element_type=jnp.float32,
    )
    t = (wsq[:, None] + sx[None, :]) - 2.0 * mm_t

    loc_min = jnp.min(t, axis=0)
    loc_arg = jnp.argmin(t, axis=0).astype(jnp.int32) + k * kt

    @pl.when(k == 0)
    def _init():
        best_ref[0, :] = loc_min
        bidx_ref[0, :] = loc_arg

    @pl.when(k > 0)
    def _update():
        prev = best_ref[0, :]
        upd = loc_min < prev
        best_ref[0, :] = jnp.where(upd, loc_min, prev)
        bidx_ref[0, :] = jnp.where(upd, loc_arg, bidx_ref[0, :])

    @pl.when(k == nk - 1)
    def _emit():
        idx_ref[0, 0, :] = bidx_ref[0, :]


def _head_argmin(emb, x, *es, bt=512, kt=1024):
    b, d = x.shape
    kk = emb.shape[0]
    nb = b // bt
    nk = kk // kt
    nsub = len(es)

    in_specs = [pl.BlockSpec((bt, d), lambda i, k: (i, 0))]
    for _ in range(nsub):
        in_specs.append(pl.BlockSpec((bt, d), lambda i, k: (i, 0)))
    in_specs.append(pl.BlockSpec((kt, d), lambda i, k: (k, 0)))

    out = pl.pallas_call(
        functools.partial(_head_body, nsub, nk),
        grid=(nb, nk),
        in_specs=in_specs,
        out_specs=pl.BlockSpec((1, 1, bt), lambda i, k: (i, 0, 0)),
        out_shape=jax.ShapeDtypeStruct((nb, 1, bt), jnp.int32),
        scratch_shapes=[
            pltpu.VMEM((1, bt), jnp.float32),
            pltpu.VMEM((1, bt), jnp.int32),
        ],
        compiler_params=pltpu.CompilerParams(
            dimension_semantics=("parallel", "arbitrary"),
        ),
    )(x, *es, emb)
    return out.reshape(b)


def _sc_gather(table, idx):
    kk, d = table.shape
    b = idx.shape[0]
    info = plsc.get_sparse_core_info()
    nw = info.num_cores * info.num_subcores
    b_per_w = b // nw
    mesh = plsc.VectorSubcoreMesh(core_axis_name="c", subcore_axis_name="s")

    @functools.partial(
        pl.kernel,
        mesh=mesh,
        out_type=jax.ShapeDtypeStruct((b, d), jnp.float32),
        scratch_types=[
            pltpu.VMEM((b_per_w,), jnp.int32),
            pltpu.VMEM((b_per_w, d), jnp.float32),
            pltpu.SemaphoreType.DMA,
        ],
    )
    def gather(table_hbm, idx_hbm, out_hbm, idx_v, rows_v, sem):
        wid = lax.axis_index("s") * info.num_cores + lax.axis_index("c")
        base = wid * b_per_w
        pltpu.sync_copy(idx_hbm.at[pl.ds(base, b_per_w)], idx_v)
        pltpu.async_copy(table_hbm.at[idx_v], rows_v, sem).wait()
        pltpu.sync_copy(rows_v, out_hbm.at[pl.ds(base, b_per_w)])

    return gather(table, idx)


def _combine_body(e0_ref, e1_ref, e2_ref, out_ref):
    out_ref[...] = e0_ref[...] + e1_ref[...] + e2_ref[...]


def _combine(e0, e1, e2, bt=512):
    b, d = e0.shape
    nb = b // bt
    spec = pl.BlockSpec((bt, d), lambda i: (i, 0))
    return pl.pallas_call(
        _combine_body,
        grid=(nb,),
        in_specs=[spec, spec, spec],
        out_specs=spec,
        out_shape=jax.ShapeDtypeStruct((b, d), jnp.float32),
    )(e0, e1, e2)


def kernel(inputs, emb0, emb1, emb2):
    b = inputs.shape[0]
    x0 = inputs[:, 0, :]

    idx0 = _head_argmin(emb0, x0)
    e0 = _sc_gather(emb0, idx0)
    idx1 = _head_argmin(emb1, x0, e0)
    e1 = _sc_gather(emb1, idx1)
    idx2 = _head_argmin(emb2, x0, e0, e1)
    e2 = _sc_gather(emb2, idx2)

    quantized = _combine(e0, e1, e2).reshape(b, 1, inputs.shape[2])
    codes = jnp.stack([idx0, idx1, idx2], axis=1)
    return quantized, codes
